# baseline (speedup 1.0000x reference)
import jax
import jax.numpy as jnp
from jax import lax
from jax.experimental import pallas as pl
from jax.experimental.pallas import tpu as pltpu
from jax.experimental.pallas import tpu_sc as plsc

EMB = 64
_NC = 2
_NS = 16
NW = _NC * _NS
G = 128


def _emb_body(x_hbm, comb_hbm, out_hbm, idx_v, gbuf, sbuf, sem):
    ng = x_hbm.shape[0] // NW
    wid = lax.axis_index("s") * _NC + lax.axis_index("c")
    pltpu.sync_copy(x_hbm.at[pl.ds(wid * ng, ng)], idx_v)
    base = wid * ng * G

    @pl.loop(0, ng)
    def _group(g):
        pltpu.async_copy(comb_hbm.at[idx_v.at[g]], gbuf, sem).wait()

        @pl.loop(0, G)
        def _row(j):
            for c in range(EMB // 16):
                s = pl.ds(c * 16, 16)
                sbuf[j, s] = gbuf[j, s] + gbuf[j, pl.ds(EMB + c * 16, 16)]

        pltpu.sync_copy(sbuf, out_hbm.at[pl.ds(base + g * G, G)])


def kernel(x, word_table, pe_table):
    b, s = x.shape
    n = b * s
    xg = x.reshape(n // G, G)
    comb = jnp.concatenate([word_table, pe_table], axis=1)
    mesh = plsc.VectorSubcoreMesh(core_axis_name="c", subcore_axis_name="s")
    out = pl.kernel(
        _emb_body,
        out_type=jax.ShapeDtypeStruct((n, EMB), jnp.float32),
        mesh=mesh,
        scratch_types=[
            pltpu.VMEM((n // G // NW, G), jnp.int32),
            pltpu.VMEM((G, 2 * EMB), jnp.float32),
            pltpu.VMEM((G, EMB), jnp.float32),
            pltpu.SemaphoreType.DMA,
        ],
    )(xg, comb)
    return out.reshape(b, s, EMB)
